# baseline (speedup 1.0000x reference)
import functools

import jax
import jax.numpy as jnp
from jax import lax
from jax.experimental import pallas as pl
from jax.experimental.pallas import tpu as pltpu
from jax.experimental.pallas import tpu_sc as plsc

_N = 4096
_LO, _HI = 1024, 3072
_NC, _NS = 2, 16
_NW = _NC * _NS
_FRPW = (_HI - _LO) // _NW
_FCH = 16

_mesh = plsc.VectorSubcoreMesh(core_axis_name="c", subcore_axis_name="s")


@functools.partial(
    pl.kernel,
    out_type=jax.ShapeDtypeStruct((1, _N, _N), jnp.float32),
    mesh=_mesh,
    scratch_types=[
        pltpu.VMEM((_FCH, _HI - _LO), jnp.float32),
        pltpu.VMEM((16,), jnp.float32),
        pltpu.SemaphoreType.DMA,
    ],
)
def _sc_fill(bt_hbm, o_hbm, fbuf, btv, sem):
    wid = lax.axis_index("s") * _NC + lax.axis_index("c")
    base = _LO + wid * _FRPW

    pltpu.sync_copy(bt_hbm, btv)
    splat = btv[...]

    def _fill(c, carry):
        for rr in range(_FCH):
            fbuf[rr, pl.ds(c * 16, 16)] = splat
        return carry

    lax.fori_loop(0, (_HI - _LO) // 16, _fill, 0)

    copies = [
        pltpu.make_async_copy(
            fbuf, o_hbm.at[0, pl.ds(base + k * _FCH, _FCH), pl.ds(_LO, _HI - _LO)],
            sem)
        for k in range(_FRPW // _FCH)
    ]
    for c in copies:
        c.start()
    for c in copies:
        c.wait()


_BR, _BC = 512, 1024
_RB0, _RB1 = _LO // _BR, _HI // _BR
_CB0, _CB1 = _LO // _BC, _HI // _BC


def _park(r, c):
    interior = (r >= _RB0) & (r < _RB1) & (c >= _CB0) & (c < _CB1)
    return (0, r, jnp.where(interior, 0, c))


def _tc_body(b_ref, filled_ref, o_ref):
    r = pl.program_id(0)
    c = pl.program_id(1)
    interior = (r >= _RB0) & (r < _RB1) & (c >= _CB0) & (c < _CB1)

    @pl.when(jnp.logical_not(interior))
    def _copy():
        o_ref[...] = b_ref[...]


def _tc_exterior(B, filled):
    return pl.pallas_call(
        _tc_body,
        grid=(_N // _BR, _N // _BC),
        in_specs=[
            pl.BlockSpec((1, _BR, _BC), _park),
            pl.BlockSpec(memory_space=pl.ANY),
        ],
        out_specs=pl.BlockSpec((1, _BR, _BC), _park),
        out_shape=jax.ShapeDtypeStruct((1, _N, _N), jnp.float32),
        input_output_aliases={1: 0},
    )(B, filled)


def kernel(B, Bt):
    bt16 = jnp.broadcast_to(jnp.reshape(Bt, (1,)), (16,))
    filled = _sc_fill(bt16)
    return _tc_exterior(B, filled)


# speedup vs baseline: 1.7554x; 1.7554x over previous
import jax
import jax.numpy as jnp
from jax.experimental import pallas as pl
from jax.experimental.pallas import tpu as pltpu

_N = 4096
_LO, _HI = 1024, 3072
_BR = 512
_M0, _M1 = _LO // _BR, _HI // _BR


def _body(full_ref, left_ref, right_ref, bt_ref, o_ref):
    i = pl.program_id(0)
    in_rows = (i >= _M0) & (i < _M1)

    @pl.when(in_rows)
    def _mid():
        o_ref[:, :, : _LO] = left_ref[...]
        o_ref[:, :, _LO:_HI] = jnp.full((1, _BR, _HI - _LO), bt_ref[0, 0],
                                        jnp.float32)
        o_ref[:, :, _HI:] = right_ref[...]

    @pl.when(jnp.logical_not(in_rows))
    def _copy():
        o_ref[...] = full_ref[...]


def _full_idx(i):
    return (0, jnp.where((i >= _M0) & (i < _M1), _M0 - 1, i), 0)


def _slab_idx(col_block):
    def idx(i):
        return (0, jnp.clip(i, _M0, _M1 - 1), col_block)
    return idx


def kernel(B, Bt):
    return pl.pallas_call(
        _body,
        grid=(_N // _BR,),
        in_specs=[
            pl.BlockSpec((1, _BR, _N), _full_idx),
            pl.BlockSpec((1, _BR, _LO), _slab_idx(0)),
            pl.BlockSpec((1, _BR, _N - _HI), _slab_idx(_HI // (_N - _HI))),
            pl.BlockSpec(memory_space=pltpu.SMEM),
        ],
        out_specs=pl.BlockSpec((1, _BR, _N), lambda i: (0, i, 0)),
        out_shape=jax.ShapeDtypeStruct((1, _N, _N), jnp.float32),
    )(B, B, B, Bt)
